# Initial kernel scaffold; baseline (speedup 1.0000x reference)
import jax
import jax.numpy as jnp
from jax.experimental import pallas as pl


def kernel(frames, masks, raw):
    raise NotImplementedError("write your pallas kernel here")


# speedup vs baseline: 188.3663x; 188.3663x over previous
import jax
import jax.numpy as jnp
from jax.experimental import pallas as pl

MAX_DELTA = 24.0


def _tc_body(raw_ref, frames_ref, masks_ref, out_ref):
    x = frames_ref[0]
    m = masks_ref[0]
    delta = MAX_DELTA * jnp.tanh(raw_ref[...])
    lut = jnp.zeros((3,) + m.shape, jnp.float32)
    for k in range(5):
        sel = (m == k).astype(jnp.float32)[None]
        lut = lut + delta[k].reshape(3, 1, 1) * sel
    upd = jnp.clip(x[1] + lut, 0.0, 255.0)
    out_ref[0, 0] = x[0]
    out_ref[0, 1] = upd


def kernel(frames, masks, raw):
    B, F, C, H, W = frames.shape
    return pl.pallas_call(
        _tc_body,
        grid=(B,),
        in_specs=[
            pl.BlockSpec((5, 3), lambda b: (0, 0)),
            pl.BlockSpec((1, F, C, H, W), lambda b: (b, 0, 0, 0, 0)),
            pl.BlockSpec((1, H, W), lambda b: (b, 0, 0)),
        ],
        out_specs=pl.BlockSpec((1, F, C, H, W), lambda b: (b, 0, 0, 0, 0)),
        out_shape=jax.ShapeDtypeStruct(frames.shape, frames.dtype),
    )(raw, frames, masks)
